# baseline (speedup 1.0000x reference)
import functools

import jax
import jax.numpy as jnp
from jax import lax
from jax.experimental import pallas as pl
from jax.experimental.pallas import tpu as pltpu
from jax.experimental.pallas import tpu_sc as plsc

_EP = 128
_LP = 128


def _pad_tc(cards, mask, B, L):
    BT = 4096

    def body(c_ref, m_ref, co_ref, mo_ref):
        co_ref[:, :L] = c_ref[...]
        mo_ref[:, :L] = m_ref[...]

    return pl.pallas_call(
        body,
        grid=(B // BT,),
        in_specs=[
            pl.BlockSpec((BT, L), lambda i: (i, 0)),
            pl.BlockSpec((BT, L), lambda i: (i, 0)),
        ],
        out_specs=[
            pl.BlockSpec((BT, _LP), lambda i: (i, 0)),
            pl.BlockSpec((BT, _LP), lambda i: (i, 0)),
        ],
        out_shape=[
            jax.ShapeDtypeStruct((B, _LP), jnp.int32),
            jax.ShapeDtypeStruct((B, _LP), jnp.float32),
        ],
    )(cards, mask)


def _hist_sc(cards_flat, mask_flat, B, L):
    info = plsc.get_sparse_core_info()
    NC, NS = info.num_cores, info.num_subcores
    NW = NC * NS
    BPW = B // NW
    CH = 64
    NCHUNK = BPW // CH

    mesh = plsc.VectorSubcoreMesh(core_axis_name="c", subcore_axis_name="s")

    @functools.partial(
        pl.kernel,
        out_type=jax.ShapeDtypeStruct((B * _EP,), jnp.float32),
        mesh=mesh,
        compiler_params=pltpu.CompilerParams(needs_layout_passes=False,
                                             use_tc_tiling_on_sc=True),
        scratch_types=[
            pltpu.VMEM((CH, L), jnp.int32),
            pltpu.VMEM((CH, L), jnp.int32),
            pltpu.VMEM((CH, L), jnp.float32),
            pltpu.VMEM((CH, L), jnp.float32),
            pltpu.VMEM((BPW * _EP // 2,), jnp.float32),
            pltpu.VMEM((BPW * _EP // 2,), jnp.float32),
            pltpu.SemaphoreType.DMA,
            pltpu.SemaphoreType.DMA,
            pltpu.SemaphoreType.DMA,
            pltpu.SemaphoreType.DMA,
            pltpu.SemaphoreType.DMA,
        ],
    )
    def hist(cards_hbm, mask_hbm, w_hbm,
             cards_v0, cards_v1, mask_v0, mask_v1, acc_a, acc_b,
             sem_c0, sem_c1, sem_m0, sem_m1, sem_out):
        wid = lax.axis_index("s") * NC + lax.axis_index("c")
        base = wid * BPW
        cbufs = (cards_v0, cards_v1)
        mbufs = (mask_v0, mask_v1)
        csems = (sem_c0, sem_c1)
        msems = (sem_m0, sem_m1)

        def start(ch):
            slot = ch % 2
            sl = pl.ds(base + ch * CH, CH)
            hc = pltpu.async_copy(cards_hbm.at[sl, :], cbufs[slot], csems[slot])
            hm = pltpu.async_copy(mask_hbm.at[sl, :], mbufs[slot], msems[slot])
            return hc, hm

        pending = start(0)

        zeros16 = jnp.zeros((16,), jnp.float32)

        def zbody(i, _):
            for j in range(8):
                acc_a[pl.ds((i * 8 + j) * 16, 16)] = zeros16
                acc_b[pl.ds((i * 8 + j) * 16, 16)] = zeros16
            return 0

        lax.fori_loop(0, BPW * _EP // (2 * 16 * 8), zbody, 0)

        lanes = lax.iota(jnp.int32, 16)
        H = CH // 2

        for ch in range(NCHUNK):
            slot = ch % 2
            hc, hm = pending
            hc.wait()
            hm.wait()
            if ch + 1 < NCHUNK:
                pending = start(ch + 1)
            cards_v = cbufs[slot]
            mask_v = mbufs[slot]

            def gbody(g, _, cards_v=cards_v, mask_v=mask_v, ch=ch):
                rows_a = g * 16 + lanes
                rows_b = rows_a + H
                dst_a = (ch * H) * _EP + rows_a * _EP
                dst_b = (ch * H) * _EP + (rows_b - H) * _EP

                def lbody(l, _):
                    lv = l + lanes
                    lv = jnp.where(lv >= L, lv - L, lv)
                    ca = plsc.load_gather(cards_v, [rows_a, lv])
                    wa = plsc.load_gather(mask_v, [rows_a, lv])
                    cb = plsc.load_gather(cards_v, [rows_b, lv])
                    wb = plsc.load_gather(mask_v, [rows_b, lv])
                    plsc.addupdate_scatter(acc_a, [dst_a + ca], wa)
                    plsc.addupdate_scatter(acc_b, [dst_b + cb], wb)
                    return 0

                lax.fori_loop(0, L, lbody, 0)
                return 0

            lax.fori_loop(0, H // 16, gbody, 0)

        handles = []
        for ch in range(NCHUNK):
            handles.append(pltpu.async_copy(
                acc_a.at[pl.ds(ch * H * _EP, H * _EP)],
                w_hbm.at[pl.ds((base + ch * CH) * _EP, H * _EP)], sem_out))
            handles.append(pltpu.async_copy(
                acc_b.at[pl.ds(ch * H * _EP, H * _EP)],
                w_hbm.at[pl.ds((base + ch * CH + H) * _EP, H * _EP)], sem_out))
        for h in handles:
            h.wait()

    return hist(cards_flat, mask_flat)


def _matmul_tc(w, e_pad, B, D):
    BT = 2048

    def body(w_ref, e_ref, o_ref):
        o_ref[...] = jnp.dot(w_ref[...], e_ref[...],
                             preferred_element_type=jnp.float32)

    return pl.pallas_call(
        body,
        grid=(B // BT,),
        in_specs=[
            pl.BlockSpec((BT, _EP), lambda i: (i, 0)),
            pl.BlockSpec((_EP, D), lambda i: (0, 0)),
        ],
        out_specs=pl.BlockSpec((BT, D), lambda i: (i, 0)),
        out_shape=jax.ShapeDtypeStruct((B, D), jnp.float32),
    )(w, e_pad)


def kernel(cards, mask, embedding):
    B, L = cards.shape
    E, D = embedding.shape
    w = _hist_sc(cards, mask, B, L).reshape(B, _EP)
    e_pad = jnp.zeros((_EP, D), jnp.float32).at[:E, :].set(embedding)
    return _matmul_tc(w, e_pad, B, D)


# speedup vs baseline: 1.0264x; 1.0264x over previous
import functools

import jax
import jax.numpy as jnp
from jax import lax
from jax.experimental import pallas as pl
from jax.experimental.pallas import tpu as pltpu
from jax.experimental.pallas import tpu_sc as plsc

_EP = 128
_LP = 128


def _pad_tc(cards, mask, B, L):
    BT = 4096

    def body(c_ref, m_ref, co_ref, mo_ref):
        co_ref[:, :L] = c_ref[...]
        mo_ref[:, :L] = m_ref[...]

    return pl.pallas_call(
        body,
        grid=(B // BT,),
        in_specs=[
            pl.BlockSpec((BT, L), lambda i: (i, 0)),
            pl.BlockSpec((BT, L), lambda i: (i, 0)),
        ],
        out_specs=[
            pl.BlockSpec((BT, _LP), lambda i: (i, 0)),
            pl.BlockSpec((BT, _LP), lambda i: (i, 0)),
        ],
        out_shape=[
            jax.ShapeDtypeStruct((B, _LP), jnp.int32),
            jax.ShapeDtypeStruct((B, _LP), jnp.float32),
        ],
    )(cards, mask)


def _hist_sc(cards_flat, mask_flat, B, L):
    info = plsc.get_sparse_core_info()
    NC, NS = info.num_cores, info.num_subcores
    NW = NC * NS
    BPW = B // NW
    CH = 64
    NCHUNK = BPW // CH

    mesh = plsc.VectorSubcoreMesh(core_axis_name="c", subcore_axis_name="s")

    @functools.partial(
        pl.kernel,
        out_type=jax.ShapeDtypeStruct((B * _EP,), jnp.float32),
        mesh=mesh,
        compiler_params=pltpu.CompilerParams(needs_layout_passes=False,
                                             use_tc_tiling_on_sc=True),
        scratch_types=[
            pltpu.VMEM((CH, L), jnp.int32),
            pltpu.VMEM((CH, L), jnp.int32),
            pltpu.VMEM((CH, L), jnp.float32),
            pltpu.VMEM((CH, L), jnp.float32),
            pltpu.VMEM((BPW * _EP // 2,), jnp.float32),
            pltpu.VMEM((BPW * _EP // 2,), jnp.float32),
            pltpu.SemaphoreType.DMA,
            pltpu.SemaphoreType.DMA,
            pltpu.SemaphoreType.DMA,
            pltpu.SemaphoreType.DMA,
            pltpu.SemaphoreType.DMA,
        ],
    )
    def hist(cards_hbm, mask_hbm, w_hbm,
             cards_v0, cards_v1, mask_v0, mask_v1, acc_a, acc_b,
             sem_c0, sem_c1, sem_m0, sem_m1, sem_out):
        wid = lax.axis_index("s") * NC + lax.axis_index("c")
        base = wid * BPW
        cbufs = (cards_v0, cards_v1)
        mbufs = (mask_v0, mask_v1)
        csems = (sem_c0, sem_c1)
        msems = (sem_m0, sem_m1)

        def start(ch):
            slot = ch % 2
            sl = pl.ds(base + ch * CH, CH)
            hc = pltpu.async_copy(cards_hbm.at[sl, :], cbufs[slot], csems[slot])
            hm = pltpu.async_copy(mask_hbm.at[sl, :], mbufs[slot], msems[slot])
            return hc, hm

        pending = start(0)

        zeros16 = jnp.zeros((16,), jnp.float32)

        def zbody(i, _):
            for j in range(4):
                acc_a[pl.ds(i * _EP + j * 16, 16)] = zeros16
                acc_b[pl.ds(i * _EP + j * 16, 16)] = zeros16
            return 0

        lax.fori_loop(0, BPW // 2, zbody, 0)

        lanes = lax.iota(jnp.int32, 16)
        H = CH // 2

        for ch in range(NCHUNK):
            slot = ch % 2
            hc, hm = pending
            hc.wait()
            hm.wait()
            if ch + 1 < NCHUNK:
                pending = start(ch + 1)
            cards_v = cbufs[slot]
            mask_v = mbufs[slot]

            def gbody(g, _, cards_v=cards_v, mask_v=mask_v, ch=ch):
                rows_a = g * 16 + lanes
                rows_b = rows_a + H
                dst_a = (ch * H) * _EP + rows_a * _EP
                dst_b = (ch * H) * _EP + (rows_b - H) * _EP

                def lbody(l, _):
                    lv = l + lanes
                    lv = jnp.where(lv >= L, lv - L, lv)
                    ca = plsc.load_gather(cards_v, [rows_a, lv])
                    wa = plsc.load_gather(mask_v, [rows_a, lv])
                    cb = plsc.load_gather(cards_v, [rows_b, lv])
                    wb = plsc.load_gather(mask_v, [rows_b, lv])
                    plsc.addupdate_scatter(acc_a, [dst_a + ca], wa)
                    plsc.addupdate_scatter(acc_b, [dst_b + cb], wb)
                    return 0

                lax.fori_loop(0, L, lbody, 0)
                return 0

            lax.fori_loop(0, H // 16, gbody, 0)

        handles = []
        for ch in range(NCHUNK):
            handles.append(pltpu.async_copy(
                acc_a.at[pl.ds(ch * H * _EP, H * _EP)],
                w_hbm.at[pl.ds((base + ch * CH) * _EP, H * _EP)], sem_out))
            handles.append(pltpu.async_copy(
                acc_b.at[pl.ds(ch * H * _EP, H * _EP)],
                w_hbm.at[pl.ds((base + ch * CH + H) * _EP, H * _EP)], sem_out))
        for h in handles:
            h.wait()

    return hist(cards_flat, mask_flat)


def _matmul_tc(w, e_pad, B, D):
    BT = 2048

    def body(w_ref, e_ref, o_ref):
        o_ref[...] = jnp.dot(w_ref[:, :64], e_ref[...],
                             preferred_element_type=jnp.float32)

    return pl.pallas_call(
        body,
        grid=(B // BT,),
        in_specs=[
            pl.BlockSpec((BT, _EP), lambda i: (i, 0)),
            pl.BlockSpec((64, D), lambda i: (0, 0)),
        ],
        out_specs=pl.BlockSpec((BT, D), lambda i: (i, 0)),
        out_shape=jax.ShapeDtypeStruct((B, D), jnp.float32),
    )(w, e_pad)


def kernel(cards, mask, embedding):
    B, L = cards.shape
    E, D = embedding.shape
    w = _hist_sc(cards, mask, B, L).reshape(B, _EP)
    e_pad = jnp.zeros((64, D), jnp.float32).at[:E, :].set(embedding)
    return _matmul_tc(w, e_pad, B, D)
